# baseline (device time: 21198 ns/iter reference)
import jax
import jax.numpy as jnp
from jax import lax
from jax.experimental import pallas as pl
from jax.experimental.pallas import tpu as pltpu

N_SPLIT = 2


def kernel(x, dy):
    k_per, m = x.shape
    _, n = dy.shape
    m_half = m // 2
    n_chunk = n // N_SPLIT

    def body(x_ref, dy_ref, out_ref, send_buf, recv_buf, send_sems, recv_sems):
        my_x = lax.axis_index("x")
        my_y = lax.axis_index("y")
        my_z = lax.axis_index("z")
        partner = (my_x, my_y, 1 - my_z)

        barrier_sem = pltpu.get_barrier_semaphore()
        pl.semaphore_signal(
            barrier_sem, inc=1,
            device_id=partner, device_id_type=pl.DeviceIdType.MESH,
        )
        pl.semaphore_wait(barrier_sem, 1)

        dyv = dy_ref[...].astype(jnp.bfloat16)

        def half_partial(lo):
            xv = x_ref[:, lo:lo + m_half].astype(jnp.bfloat16)
            return lax.dot_general(
                xv, dyv, (((0,), (0,)), ((), ())),
                preferred_element_type=jnp.float32,
            )

        @pl.when(my_z == 0)
        def _():
            send_buf[...] = half_partial(m_half).astype(jnp.bfloat16)

        @pl.when(my_z == 1)
        def _():
            send_buf[...] = half_partial(0).astype(jnp.bfloat16)

        rdmas = []
        for c in range(N_SPLIT):
            rdma = pltpu.make_async_remote_copy(
                src_ref=send_buf.at[:, pl.ds(c * n_chunk, n_chunk)],
                dst_ref=recv_buf.at[:, pl.ds(c * n_chunk, n_chunk)],
                send_sem=send_sems.at[c],
                recv_sem=recv_sems.at[c],
                device_id=partner,
                device_id_type=pl.DeviceIdType.MESH,
            )
            rdma.start()
            rdmas.append(rdma)

        @pl.when(my_z == 0)
        def _():
            out_ref[...] = half_partial(0)

        @pl.when(my_z == 1)
        def _():
            out_ref[...] = half_partial(m_half)

        for rdma in rdmas:
            rdma.wait()
        out_ref[...] += recv_buf[...].astype(jnp.float32)

    return pl.pallas_call(
        body,
        out_shape=jax.ShapeDtypeStruct((m_half, n), jnp.float32),
        in_specs=[
            pl.BlockSpec(memory_space=pltpu.VMEM),
            pl.BlockSpec(memory_space=pltpu.VMEM),
        ],
        out_specs=pl.BlockSpec(memory_space=pltpu.VMEM),
        scratch_shapes=[
            pltpu.VMEM((m_half, n), jnp.bfloat16),
            pltpu.VMEM((m_half, n), jnp.bfloat16),
            pltpu.SemaphoreType.DMA((N_SPLIT,)),
            pltpu.SemaphoreType.DMA((N_SPLIT,)),
        ],
        compiler_params=pltpu.CompilerParams(collective_id=0),
    )(x, dy)


# device time: 20748 ns/iter; 1.0217x vs baseline; 1.0217x over previous
import jax
import jax.numpy as jnp
from jax import lax
from jax.experimental import pallas as pl
from jax.experimental.pallas import tpu as pltpu

N_CHUNKS = 4


def kernel(x, dy):
    k_per, m = x.shape
    _, n = dy.shape
    m_half = m // 2
    n_chunk = n // N_CHUNKS

    def body(x_ref, dy_ref, out_ref, send_buf, recv_buf, send_sems, recv_sems):
        my_x = lax.axis_index("x")
        my_y = lax.axis_index("y")
        my_z = lax.axis_index("z")
        partner = (my_x, my_y, 1 - my_z)

        barrier_sem = pltpu.get_barrier_semaphore()
        pl.semaphore_signal(
            barrier_sem, inc=1,
            device_id=partner, device_id_type=pl.DeviceIdType.MESH,
        )
        pl.semaphore_wait(barrier_sem, 1)

        def half_chunk(lo, c):
            xv = x_ref[:, lo:lo + m_half].astype(jnp.bfloat16)
            dv = dy_ref[:, c * n_chunk:(c + 1) * n_chunk].astype(jnp.bfloat16)
            return lax.dot_general(
                xv, dv, (((0,), (0,)), ((), ())),
                preferred_element_type=jnp.float32,
            )

        rdmas = []
        for c in range(N_CHUNKS):
            @pl.when(my_z == 0)
            def _(c=c):
                send_buf[:, c * n_chunk:(c + 1) * n_chunk] = (
                    half_chunk(m_half, c).astype(jnp.bfloat16))

            @pl.when(my_z == 1)
            def _(c=c):
                send_buf[:, c * n_chunk:(c + 1) * n_chunk] = (
                    half_chunk(0, c).astype(jnp.bfloat16))

            rdma = pltpu.make_async_remote_copy(
                src_ref=send_buf.at[:, pl.ds(c * n_chunk, n_chunk)],
                dst_ref=recv_buf.at[:, pl.ds(c * n_chunk, n_chunk)],
                send_sem=send_sems.at[c],
                recv_sem=recv_sems.at[c],
                device_id=partner,
                device_id_type=pl.DeviceIdType.MESH,
            )
            rdma.start()
            rdmas.append(rdma)

        for c in range(N_CHUNKS):
            rdmas[c].wait()

            @pl.when(my_z == 0)
            def _(c=c):
                out_ref[:, c * n_chunk:(c + 1) * n_chunk] = (
                    half_chunk(0, c)
                    + recv_buf[:, c * n_chunk:(c + 1) * n_chunk].astype(
                        jnp.float32))

            @pl.when(my_z == 1)
            def _(c=c):
                out_ref[:, c * n_chunk:(c + 1) * n_chunk] = (
                    half_chunk(m_half, c)
                    + recv_buf[:, c * n_chunk:(c + 1) * n_chunk].astype(
                        jnp.float32))

    return pl.pallas_call(
        body,
        out_shape=jax.ShapeDtypeStruct((m_half, n), jnp.float32),
        in_specs=[
            pl.BlockSpec(memory_space=pltpu.VMEM),
            pl.BlockSpec(memory_space=pltpu.VMEM),
        ],
        out_specs=pl.BlockSpec(memory_space=pltpu.VMEM),
        scratch_shapes=[
            pltpu.VMEM((m_half, n), jnp.bfloat16),
            pltpu.VMEM((m_half, n), jnp.bfloat16),
            pltpu.SemaphoreType.DMA((N_CHUNKS,)),
            pltpu.SemaphoreType.DMA((N_CHUNKS,)),
        ],
        compiler_params=pltpu.CompilerParams(collective_id=0),
    )(x, dy)
